# baseline (speedup 1.0000x reference)
import functools

import jax
import jax.numpy as jnp
from jax import lax
from jax.experimental import pallas as pl
from jax.experimental.pallas import tpu as pltpu
from jax.experimental.pallas import tpu_sc as plsc

V = 100000
H = 32
B = 1024
W = 20

NC = 2
NS = 16
NW = NC * NS
LANES = 16
N_IDX = B * W
BCHUNKS = B // LANES

VT = 4096


def _sc_pool(ctx_hbm, emt_hbm, out_hbm, idx_v, row_v, acc_v, sem):
    h = lax.axis_index("s") * NC + lax.axis_index("c")
    c1 = pltpu.async_copy(ctx_hbm, idx_v, sem)
    c2 = pltpu.async_copy(emt_hbm.at[h], row_v, sem)
    c1.wait()
    c2.wait()
    inv_w = jnp.float32(1.0 / W)

    def chunk_body(c, _):
        base = c * LANES
        acc = jnp.zeros((LANES,), jnp.float32)
        for w in range(W):
            idx = idx_v[pl.ds(w * B + base, LANES)]
            acc = acc + plsc.load_gather(row_v, [idx])
        acc_v[pl.ds(base, LANES)] = acc * inv_w
        return _

    lax.fori_loop(0, BCHUNKS, chunk_body, 0, unroll=2)
    pltpu.sync_copy(acc_v, out_hbm.at[h])


@functools.lru_cache(maxsize=1)
def _sc_pool_call():
    return functools.partial(
        pl.kernel,
        out_type=jax.ShapeDtypeStruct((H, B), jnp.float32),
        mesh=plsc.VectorSubcoreMesh(core_axis_name="c", subcore_axis_name="s"),
        scratch_types=[
            pltpu.VMEM((N_IDX,), jnp.int32),
            pltpu.VMEM((V,), jnp.float32),
            pltpu.VMEM((B,), jnp.float32),
            pltpu.SemaphoreType.DMA,
        ],
        compiler_params=pltpu.CompilerParams(
            use_tc_tiling_on_sc=False, needs_layout_passes=False
        ),
    )(_sc_pool)


def _mm_body(wt_ref, avgt_ref, b_ref, o_ref):
    yt = lax.dot_general(
        wt_ref[...],
        avgt_ref[...],
        dimension_numbers=(((0,), (0,)), ((), ())),
        preferred_element_type=jnp.float32,
    )
    b_row = b_ref[...].reshape(1, VT)
    bias = lax.dot_general(
        b_row,
        jnp.ones((1, B), jnp.float32),
        dimension_numbers=(((0,), (0,)), ((), ())),
        preferred_element_type=jnp.float32,
    )
    o_ref[...] = yt + bias


def kernel(contexts, in_emb, out_W, out_b):
    ctx_wmajor = contexts.T.reshape(N_IDX).astype(jnp.int32)
    avgt = _sc_pool_call()(ctx_wmajor, in_emb.T)
    yt = pl.pallas_call(
        _mm_body,
        grid=(pl.cdiv(V, VT),),
        in_specs=[
            pl.BlockSpec((H, VT), lambda i: (0, i)),
            pl.BlockSpec((H, B), lambda i: (0, 0)),
            pl.BlockSpec((VT,), lambda i: (i,)),
        ],
        out_specs=pl.BlockSpec((VT, B), lambda i: (i, 0)),
        out_shape=jax.ShapeDtypeStruct((V, B), jnp.float32),
        compiler_params=pltpu.CompilerParams(
            dimension_semantics=("parallel",),
        ),
    )(out_W.T, avgt, out_b)
    return yt.T
